# baseline (speedup 1.0000x reference)
import functools

import jax
import jax.numpy as jnp
from jax import lax
from jax.experimental import pallas as pl
from jax.experimental.pallas import tpu as pltpu
from jax.experimental.pallas import tpu_sc as plsc

NC = 2
NS = 16
NW = NC * NS

W = 4


def _make_gather(b, s, d):
    rows_per_w = b // NW

    @functools.partial(
        pl.kernel,
        out_type=jax.ShapeDtypeStruct((b, s, d), jnp.float32),
        mesh=plsc.VectorSubcoreMesh(core_axis_name="c", subcore_axis_name="s"),
        scratch_types=(
            [pltpu.VMEM((rows_per_w, s), jnp.int32),
             pltpu.VMEM((W, s, d), jnp.float32)]
            + [pltpu.SemaphoreType.DMA] * (2 * W)
        ),
        compiler_params=pltpu.CompilerParams(use_tc_tiling_on_sc=False),
    )
    def body(table_hbm, idx_hbm, out_hbm, idx_v, rows_v, *sems):
        wid = lax.axis_index("s") * NC + lax.axis_index("c")
        base = wid * rows_per_w
        pltpu.sync_copy(idx_hbm.at[pl.ds(base, rows_per_w)], idx_v)
        sems_g = sems[:W]
        sems_w = sems[W:]

        def fire_gather(r, slot):
            pltpu.async_copy(
                table_hbm.at[idx_v.at[r]], rows_v.at[slot], sems_g[slot]
            )

        def wait_gather(slot):
            pltpu.make_async_copy(
                table_hbm.at[idx_v.at[0]], rows_v.at[slot], sems_g[slot]
            ).wait()

        def fire_write(r, slot):
            pltpu.async_copy(
                rows_v.at[slot], out_hbm.at[base + r], sems_w[slot]
            )

        def wait_write(slot):
            pltpu.make_async_copy(
                rows_v.at[slot], out_hbm.at[base], sems_w[slot]
            ).wait()

        fire_gather(0, 0)
        for r in range(W - 1):
            fire_gather(r + 1, r + 1)
            wait_gather(r)
            fire_write(r, r)

        @pl.loop(0, rows_per_w - W, step=W)
        def _main(j):
            for b in range(W):
                r = j + b + (W - 1)
                slot = (b + W - 1) % W
                wait_write(b)
                fire_gather(r + 1, b)
                wait_gather(slot)
                fire_write(r, slot)

        wait_gather((rows_per_w - 1) % W)
        fire_write(rows_per_w - 1, (rows_per_w - 1) % W)
        for slot in range(W):
            wait_write(slot)

    return body


def kernel(seq, embedding_weight):
    b, s = seq.shape
    _, d = embedding_weight.shape
    return _make_gather(b, s, d)(embedding_weight, seq.astype(jnp.int32))
